# baseline (speedup 1.0000x reference)
import jax
import jax.numpy as jnp
from jax.experimental import pallas as pl

_N_CLASSES = 1000
_B, _T = 4096, 26
_B_BLK = 512


def _onehot_block(idx_ref, out_ref):
    idx = idx_ref[...]
    iota = jax.lax.broadcasted_iota(jnp.int32, (1, _N_CLASSES, _B_BLK), 1)
    out_ref[...] = (iota == idx).astype(jnp.float32)


def kernel(input):
    idx_t = input.astype(jnp.int32).T.reshape(_T, 1, _B)
    out = pl.pallas_call(
        _onehot_block,
        grid=(_T, _B // _B_BLK),
        in_specs=[pl.BlockSpec((1, 1, _B_BLK), lambda t, j: (t, 0, j))],
        out_specs=pl.BlockSpec((1, _N_CLASSES, _B_BLK), lambda t, j: (t, 0, j)),
        out_shape=jax.ShapeDtypeStruct((_T, _N_CLASSES, _B), jnp.float32),
    )(idx_t)
    return out.transpose(2, 0, 1)


# speedup vs baseline: 1.3198x; 1.3198x over previous
import jax
import jax.numpy as jnp
from jax.experimental import pallas as pl

_N_CLASSES = 1000
_B, _T = 4096, 26
_B_BLK = 2048


def _onehot_block(idx_ref, out_ref):
    t = pl.program_id(0)
    idx = idx_ref[pl.ds(t, 1), :]
    iota = jax.lax.broadcasted_iota(jnp.int32, (1, _N_CLASSES, _B_BLK), 1)
    out_ref[...] = (iota == idx[:, None, :]).astype(jnp.float32)


def kernel(input):
    idx_t = input.astype(jnp.int32).T
    out = pl.pallas_call(
        _onehot_block,
        grid=(_T, _B // _B_BLK),
        in_specs=[pl.BlockSpec((_T, _B_BLK), lambda t, j: (0, j))],
        out_specs=pl.BlockSpec((1, _N_CLASSES, _B_BLK), lambda t, j: (t, 0, j)),
        out_shape=jax.ShapeDtypeStruct((_T, _N_CLASSES, _B), jnp.float32),
    )(idx_t)
    return out.transpose(2, 0, 1)


# speedup vs baseline: 1.3351x; 1.0116x over previous
import jax
import jax.numpy as jnp
from jax.experimental import pallas as pl

_N_CLASSES = 1000
_B, _T = 4096, 26
_B_BLK = 2048
_T_BLK = 2


def _onehot_block(idx_ref, out_ref):
    idx = idx_ref[...]
    iota = jax.lax.broadcasted_iota(
        jnp.int32, (_T_BLK, _N_CLASSES, _B_BLK), 1
    )
    out_ref[...] = (iota == idx).astype(jnp.float32)


def kernel(input):
    idx_t = input.astype(jnp.int32).T.reshape(_T, 1, _B)
    out = pl.pallas_call(
        _onehot_block,
        grid=(_T // _T_BLK, _B // _B_BLK),
        in_specs=[pl.BlockSpec((_T_BLK, 1, _B_BLK), lambda t, j: (t, 0, j))],
        out_specs=pl.BlockSpec(
            (_T_BLK, _N_CLASSES, _B_BLK), lambda t, j: (t, 0, j)
        ),
        out_shape=jax.ShapeDtypeStruct((_T, _N_CLASSES, _B), jnp.float32),
    )(idx_t)
    return out.transpose(2, 0, 1)


# speedup vs baseline: 1.3527x; 1.0131x over previous
import jax
import jax.numpy as jnp
from jax.experimental import pallas as pl

_N_CLASSES = 1000
_B, _T = 4096, 26
_B_BLK = 2048
_T_BLK = 1


def _onehot_block(idx_ref, out_ref):
    idx = idx_ref[...]
    iota = jax.lax.broadcasted_iota(
        jnp.int32, (_T_BLK, _N_CLASSES, _B_BLK), 1
    )
    out_ref[...] = (iota == idx).astype(jnp.float32)


def kernel(input):
    idx_t = input.astype(jnp.int32).T.reshape(_T, 1, _B)
    out = pl.pallas_call(
        _onehot_block,
        grid=(_T // _T_BLK, _B // _B_BLK),
        in_specs=[pl.BlockSpec((_T_BLK, 1, _B_BLK), lambda t, j: (t, 0, j))],
        out_specs=pl.BlockSpec(
            (_T_BLK, _N_CLASSES, _B_BLK), lambda t, j: (t, 0, j)
        ),
        out_shape=jax.ShapeDtypeStruct((_T, _N_CLASSES, _B), jnp.float32),
    )(idx_t)
    return out.transpose(2, 0, 1)
